# baseline (speedup 1.0000x reference)
import jax
import jax.numpy as jnp
import numpy as np
from jax.experimental import pallas as pl
from jax.experimental.pallas import tpu as pltpu
from jax import shard_map
from jax.sharding import Mesh, PartitionSpec

_BI = 256


def _chamfer_block_kernel(gts_ref, predsT_ref, out_ref, colmin_ref, rowacc_ref):
    i = pl.program_id(1)
    n_i = pl.num_programs(1)

    @pl.when(i == 0)
    def _():
        rowacc_ref[0] = 0.0
        colmin_ref[...] = jnp.full_like(colmin_ref[...], jnp.inf)

    x = gts_ref[0]
    yT = predsT_ref[0]

    rx = jnp.sum(x * x, axis=1, keepdims=True)
    ry = jnp.sum(yT * yT, axis=0, keepdims=True)

    zz2 = jax.lax.dot_general(
        x * -2.0, yT, (((1,), (0,)), ((), ())),
        preferred_element_type=jnp.float32)

    p = zz2 + rx + ry

    rowacc_ref[0] += jnp.sum(jnp.min(p, axis=1))
    colmin_ref[...] = jnp.minimum(colmin_ref[...],
                                  jnp.min(p, axis=0, keepdims=True))

    @pl.when(i == n_i - 1)
    def _():
        total = rowacc_ref[0] + jnp.sum(colmin_ref[...])
        out_ref[...] = jnp.full((1, 1, 1), total, dtype=jnp.float32)


def _chamfer_pallas(preds, gts, interpret=False):
    B, N, D = preds.shape
    predsT = jnp.transpose(preds, (0, 2, 1))
    return pl.pallas_call(
        _chamfer_block_kernel,
        out_shape=jax.ShapeDtypeStruct((B, 1, 1), jnp.float32),
        grid=(B, N // _BI),
        in_specs=[
            pl.BlockSpec((1, _BI, D), lambda b, i: (b, i, 0)),
            pl.BlockSpec((1, D, N), lambda b, i: (b, 0, 0)),
        ],
        out_specs=pl.BlockSpec((1, 1, 1), lambda b, i: (b, 0, 0)),
        scratch_shapes=[
            pltpu.VMEM((1, N), jnp.float32),
            pltpu.SMEM((1,), jnp.float32),
        ],
        compiler_params=pltpu.CompilerParams(
            dimension_semantics=("parallel", "arbitrary"),
        ),
        name="chamfer_loss",
        interpret=interpret,
    )(gts, predsT)


def _chamfer(preds, gts, interpret=False):
    return jnp.sum(_chamfer_pallas(preds, gts, interpret=interpret))


def _shard_body(preds, gts):
    return jnp.sum(_chamfer_pallas(preds, gts)).reshape(1)


def kernel(preds, gts):
    B = preds.shape[0]
    devs = jax.devices()
    nd = min(len(devs), B)
    while B % nd:
        nd -= 1
    if nd > 1 and devs[0].platform == "tpu":
        mesh = Mesh(np.asarray(devs[:nd]), ("d",))
        partials = shard_map(
            _shard_body, mesh=mesh,
            in_specs=(PartitionSpec("d"), PartitionSpec("d")),
            out_specs=PartitionSpec("d"), check_vma=False,
        )(preds, gts)
        return jnp.sum(partials)
    return _chamfer(preds, gts)


# speedup vs baseline: 1.6600x; 1.6600x over previous
import jax
import jax.numpy as jnp
from jax.experimental import pallas as pl
from jax.experimental.pallas import tpu as pltpu

_BI = 256


def _bf16_split3(v):
    hi = v.astype(jnp.bfloat16).astype(jnp.float32)
    r = v - hi
    mid = r.astype(jnp.bfloat16).astype(jnp.float32)
    lo = r - mid
    return hi, mid, lo


def _chamfer_block_kernel(gts_ref, predsT_ref, out_ref,
                          yaug_ref, colmin_ref, rowacc_ref):
    i = pl.program_id(1)
    n_i = pl.num_programs(1)
    N = predsT_ref.shape[2]

    @pl.when(i == 0)
    def _():
        yT = predsT_ref[0]
        ry = jnp.sum(yT * yT, axis=0, keepdims=True)
        ry_hi, ry_mid, ry_lo = _bf16_split3(ry)
        yaug_ref[...] = jnp.concatenate(
            [yT, jnp.ones((3, N), jnp.float32), ry_hi, ry_mid, ry_lo], axis=0)
        rowacc_ref[0] = 0.0
        colmin_ref[...] = jnp.full_like(colmin_ref[...], jnp.inf)

    x = gts_ref[0]
    rx = jnp.sum(x * x, axis=1, keepdims=True)
    rx_hi, rx_mid, rx_lo = _bf16_split3(rx)
    xaug = jnp.concatenate(
        [x * -2.0, rx_hi, rx_mid, rx_lo,
         jnp.ones((x.shape[0], 3), jnp.float32)], axis=1)

    p = jax.lax.dot_general(
        xaug, yaug_ref[...], (((1,), (0,)), ((), ())),
        preferred_element_type=jnp.float32)

    rowacc_ref[0] += jnp.sum(jnp.min(p, axis=1))
    colmin_ref[...] = jnp.minimum(colmin_ref[...],
                                  jnp.min(p, axis=0, keepdims=True))

    @pl.when(i == n_i - 1)
    def _():
        total = rowacc_ref[0] + jnp.sum(colmin_ref[...])
        out_ref[...] = jnp.full((1, 1, 1), total, dtype=jnp.float32)


def _chamfer(preds, gts, interpret=False):
    B, N, D = preds.shape
    predsT = jnp.transpose(preds, (0, 2, 1))
    out = pl.pallas_call(
        _chamfer_block_kernel,
        out_shape=jax.ShapeDtypeStruct((B, 1, 1), jnp.float32),
        grid=(B, N // _BI),
        in_specs=[
            pl.BlockSpec((1, _BI, D), lambda b, i: (b, i, 0)),
            pl.BlockSpec((1, D, N), lambda b, i: (b, 0, 0)),
        ],
        out_specs=pl.BlockSpec((1, 1, 1), lambda b, i: (b, 0, 0)),
        scratch_shapes=[
            pltpu.VMEM((3 * D, N), jnp.float32),
            pltpu.VMEM((1, N), jnp.float32),
            pltpu.SMEM((1,), jnp.float32),
        ],
        compiler_params=pltpu.CompilerParams(
            dimension_semantics=("parallel", "arbitrary"),
        ),
        name="chamfer_loss",
        interpret=interpret,
    )(gts, predsT)
    return jnp.sum(out)


def kernel(preds, gts):
    return _chamfer(preds, gts)


# speedup vs baseline: 1.9231x; 1.1585x over previous
import jax
import jax.numpy as jnp
from jax.experimental import pallas as pl
from jax.experimental.pallas import tpu as pltpu

_BI = 512


def _bf16_split3(v):
    hi = v.astype(jnp.bfloat16).astype(jnp.float32)
    r = v - hi
    mid = r.astype(jnp.bfloat16).astype(jnp.float32)
    lo = r - mid
    return hi, mid, lo


def _chamfer_block_kernel(gts_ref, predsT_ref, out_ref,
                          yaug_ref, colmin_ref, rowacc_ref):
    i = pl.program_id(1)
    n_i = pl.num_programs(1)
    N = predsT_ref.shape[2]

    @pl.when(i == 0)
    def _():
        yT = predsT_ref[0]
        ry = jnp.sum(yT * yT, axis=0, keepdims=True)
        ry_hi, ry_mid, ry_lo = _bf16_split3(ry)
        yaug_ref[...] = jnp.concatenate(
            [yT, jnp.ones((3, N), jnp.float32), ry_hi, ry_mid, ry_lo], axis=0)
        rowacc_ref[0] = 0.0
        colmin_ref[...] = jnp.full_like(colmin_ref[...], jnp.inf)

    x = gts_ref[0]
    rx = jnp.sum(x * x, axis=1, keepdims=True)
    rx_hi, rx_mid, rx_lo = _bf16_split3(rx)
    xaug = jnp.concatenate(
        [x * -2.0, rx_hi, rx_mid, rx_lo,
         jnp.ones((x.shape[0], 3), jnp.float32)], axis=1)

    p = jax.lax.dot_general(
        xaug, yaug_ref[...], (((1,), (0,)), ((), ())),
        preferred_element_type=jnp.float32)

    rowacc_ref[0] += jnp.sum(jnp.min(p, axis=1))
    colmin_ref[...] = jnp.minimum(colmin_ref[...],
                                  jnp.min(p, axis=0, keepdims=True))

    @pl.when(i == n_i - 1)
    def _():
        total = rowacc_ref[0] + jnp.sum(colmin_ref[...])
        out_ref[...] = jnp.full((1, 1, 1), total, dtype=jnp.float32)


def _chamfer(preds, gts, interpret=False):
    B, N, D = preds.shape
    predsT = jnp.transpose(preds, (0, 2, 1))
    out = pl.pallas_call(
        _chamfer_block_kernel,
        out_shape=jax.ShapeDtypeStruct((B, 1, 1), jnp.float32),
        grid=(B, N // _BI),
        in_specs=[
            pl.BlockSpec((1, _BI, D), lambda b, i: (b, i, 0)),
            pl.BlockSpec((1, D, N), lambda b, i: (b, 0, 0)),
        ],
        out_specs=pl.BlockSpec((1, 1, 1), lambda b, i: (b, 0, 0)),
        scratch_shapes=[
            pltpu.VMEM((3 * D, N), jnp.float32),
            pltpu.VMEM((1, N), jnp.float32),
            pltpu.SMEM((1,), jnp.float32),
        ],
        compiler_params=pltpu.CompilerParams(
            dimension_semantics=("parallel", "arbitrary"),
        ),
        name="chamfer_loss",
        interpret=interpret,
    )(gts, predsT)
    return jnp.sum(out)


def kernel(preds, gts):
    return _chamfer(preds, gts)


# speedup vs baseline: 2.0877x; 1.0856x over previous
import jax
import jax.numpy as jnp
from jax.experimental import pallas as pl
from jax.experimental.pallas import tpu as pltpu

_BI = 1024


def _bf16_split3(v):
    hi = v.astype(jnp.bfloat16).astype(jnp.float32)
    r = v - hi
    mid = r.astype(jnp.bfloat16).astype(jnp.float32)
    lo = r - mid
    return hi, mid, lo


def _chamfer_block_kernel(gts_ref, predsT_ref, out_ref,
                          yaug_ref, colmin_ref, rowacc_ref):
    i = pl.program_id(1)
    n_i = pl.num_programs(1)
    N = predsT_ref.shape[2]

    @pl.when(i == 0)
    def _():
        yT = predsT_ref[0]
        ry = jnp.sum(yT * yT, axis=0, keepdims=True)
        ry_hi, ry_mid, ry_lo = _bf16_split3(ry)
        yaug_ref[...] = jnp.concatenate(
            [yT, jnp.ones((3, N), jnp.float32), ry_hi, ry_mid, ry_lo], axis=0)
        rowacc_ref[0] = 0.0
        colmin_ref[...] = jnp.full_like(colmin_ref[...], jnp.inf)

    x = gts_ref[0]
    rx = jnp.sum(x * x, axis=1, keepdims=True)
    rx_hi, rx_mid, rx_lo = _bf16_split3(rx)
    xaug = jnp.concatenate(
        [x * -2.0, rx_hi, rx_mid, rx_lo,
         jnp.ones((x.shape[0], 3), jnp.float32)], axis=1)

    p = jax.lax.dot_general(
        xaug, yaug_ref[...], (((1,), (0,)), ((), ())),
        preferred_element_type=jnp.float32)

    rowacc_ref[0] += jnp.sum(jnp.min(p, axis=1))
    colmin_ref[...] = jnp.minimum(colmin_ref[...],
                                  jnp.min(p, axis=0, keepdims=True))

    @pl.when(i == n_i - 1)
    def _():
        total = rowacc_ref[0] + jnp.sum(colmin_ref[...])
        out_ref[...] = jnp.full((1, 1, 1), total, dtype=jnp.float32)


def _chamfer(preds, gts, interpret=False):
    B, N, D = preds.shape
    predsT = jnp.transpose(preds, (0, 2, 1))
    out = pl.pallas_call(
        _chamfer_block_kernel,
        out_shape=jax.ShapeDtypeStruct((B, 1, 1), jnp.float32),
        grid=(B, N // _BI),
        in_specs=[
            pl.BlockSpec((1, _BI, D), lambda b, i: (b, i, 0)),
            pl.BlockSpec((1, D, N), lambda b, i: (b, 0, 0)),
        ],
        out_specs=pl.BlockSpec((1, 1, 1), lambda b, i: (b, 0, 0)),
        scratch_shapes=[
            pltpu.VMEM((3 * D, N), jnp.float32),
            pltpu.VMEM((1, N), jnp.float32),
            pltpu.SMEM((1,), jnp.float32),
        ],
        compiler_params=pltpu.CompilerParams(
            dimension_semantics=("parallel", "arbitrary"),
            vmem_limit_bytes=56 * 1024 * 1024,
        ),
        name="chamfer_loss",
        interpret=interpret,
    )(gts, predsT)
    return jnp.sum(out)


def kernel(preds, gts):
    return _chamfer(preds, gts)
